# baseline (speedup 1.0000x reference)
import functools
import math

import jax
import jax.numpy as jnp
from jax.experimental import pallas as pl
from jax.experimental.pallas import tpu as pltpu

BN_EPS = 1e-5
_C2 = 1.4426950408889634 / math.pi
_SUB = 512


def _main_kernel(xT_ref, w_ref, b_ref,
                 out_ref, ssum_ref, ssq_ref,
                 av_ref, hsq_ref, colsum_ref, deg_ref, xb_ref, xd_ref,
                 *, bn, nb):
    sub = min(_SUB, bn)
    p = pl.program_id(1)
    i = pl.program_id(2)

    @pl.when((p == 0) & (i == 0))
    def _():
        xT = xT_ref[0]
        hsq_ref[...] = 0.5 * jnp.sum(xT * xT, axis=0, keepdims=True)
        xb_ref[...] = xT.astype(jnp.bfloat16)

    @pl.when(p == 0)
    def _():
        parts = []
        for s in range(bn // sub):
            off = i * bn + s * sub
            xb_s = xb_ref[:, pl.ds(off, sub)]
            gram = jax.lax.dot_general(
                xb_s, xb_ref[...], (((0,), (0,)), ((), ())),
                preferred_element_type=jnp.float32)
            hsq_col = hsq_ref[:, pl.ds(off, sub)].T
            z = (gram - hsq_col) - hsq_ref[...]
            av = jnp.exp2(z * _C2)
            av_ref[pl.ds(off, sub), :] = av.astype(jnp.bfloat16)
            parts.append(jnp.sum(av, axis=0, keepdims=True))
        part = parts[0]
        for q in parts[1:]:
            part = part + q

        @pl.when(i == 0)
        def _():
            colsum_ref[...] = part

        @pl.when(i != 0)
        def _():
            colsum_ref[...] += part

    @pl.when(p == 1)
    def _():
        @pl.when(i == 0)
        def _():
            deg = jax.lax.rsqrt(1.0 + colsum_ref[...])
            deg_ref[...] = deg
            xd_ref[...] = (xT_ref[0] * deg).astype(jnp.bfloat16)

        psums, psqs = [], []
        for s in range(bn // sub):
            off = i * bn + s * sub
            av_col = av_ref[:, pl.ds(off, sub)]
            m1t = jax.lax.dot_general(
                xd_ref[...], av_col, (((1,), (0,)), ((), ())),
                preferred_element_type=jnp.float32)
            deg_blk = deg_ref[:, pl.ds(off, sub)]
            xT_blk = xT_ref[0, :, pl.ds(off, sub)]
            aggT = deg_blk * m1t + (deg_blk * deg_blk) * xT_blk
            avwt = jax.lax.dot_general(
                w_ref[...], aggT, (((1,), (0,)), ((), ())),
                preferred_element_type=jnp.float32) + b_ref[...]
            out_ref[0, :, s * sub:(s + 1) * sub] = avwt
            psums.append(jnp.sum(avwt, axis=1, keepdims=True))
            psqs.append(jnp.sum(avwt * avwt, axis=1, keepdims=True))
        psum = psums[0]
        psq = psqs[0]
        for q in psums[1:]:
            psum = psum + q
        for q in psqs[1:]:
            psq = psq + q

        @pl.when(i == 0)
        def _():
            ssum_ref[0] = psum
            ssq_ref[0] = psq

        @pl.when(i != 0)
        def _():
            ssum_ref[0] += psum
            ssq_ref[0] += psq


def _bn_kernel(avwt_ref, ssum_ref, ssq_ref, gamma_ref, beta_ref, out_ref,
               *, count):
    b_total = ssum_ref.shape[0]
    s = ssum_ref[0]
    q = ssq_ref[0]
    for bb in range(1, b_total):
        s = s + ssum_ref[bb]
        q = q + ssq_ref[bb]
    inv = 1.0 / count
    mean = s * inv
    var = q * inv - mean * mean
    scale = gamma_ref[...] * jax.lax.rsqrt(var + BN_EPS)
    shift = beta_ref[...] - mean * scale
    out_ref[0] = avwt_ref[0] * scale + shift


def kernel(x, W, b_lin, gamma, beta):
    b, c, h, w = x.shape
    n = h * w
    bn = min(2048, n)
    nb = n // bn

    xT = x.reshape(b, c, n)

    avwt, ssum, ssq = pl.pallas_call(
        functools.partial(_main_kernel, bn=bn, nb=nb),
        grid=(b, 2, nb),
        in_specs=[
            pl.BlockSpec((1, c, n), lambda bi, p, i: (bi, 0, 0)),
            pl.BlockSpec((c, c), lambda bi, p, i: (0, 0)),
            pl.BlockSpec((c, 1), lambda bi, p, i: (0, 0)),
        ],
        out_specs=[
            pl.BlockSpec((1, c, bn), lambda bi, p, i: (bi, 0, i * p)),
            pl.BlockSpec((1, c, 1), lambda bi, p, i: (bi, 0, 0)),
            pl.BlockSpec((1, c, 1), lambda bi, p, i: (bi, 0, 0)),
        ],
        out_shape=[
            jax.ShapeDtypeStruct((b, c, n), jnp.float32),
            jax.ShapeDtypeStruct((b, c, 1), jnp.float32),
            jax.ShapeDtypeStruct((b, c, 1), jnp.float32),
        ],
        scratch_shapes=[
            pltpu.VMEM((n, n), jnp.bfloat16),
            pltpu.VMEM((1, n), jnp.float32),
            pltpu.VMEM((1, n), jnp.float32),
            pltpu.VMEM((1, n), jnp.float32),
            pltpu.VMEM((c, n), jnp.bfloat16),
            pltpu.VMEM((c, n), jnp.bfloat16),
        ],
        compiler_params=pltpu.CompilerParams(
            dimension_semantics=("arbitrary", "arbitrary", "arbitrary"),
            vmem_limit_bytes=100 * 1024 * 1024,
        ),
    )(xT, W, b_lin[:, None])

    bn3 = min(4096, n)
    y = pl.pallas_call(
        functools.partial(_bn_kernel, count=float(b * n)),
        grid=(b, n // bn3),
        in_specs=[
            pl.BlockSpec((1, c, bn3), lambda bi, i: (bi, 0, i)),
            pl.BlockSpec((b, c, 1), lambda bi, i: (0, 0, 0)),
            pl.BlockSpec((b, c, 1), lambda bi, i: (0, 0, 0)),
            pl.BlockSpec((c, 1), lambda bi, i: (0, 0)),
            pl.BlockSpec((c, 1), lambda bi, i: (0, 0)),
        ],
        out_specs=pl.BlockSpec((1, c, bn3), lambda bi, i: (bi, 0, i)),
        out_shape=jax.ShapeDtypeStruct((b, c, n), jnp.float32),
        compiler_params=pltpu.CompilerParams(
            dimension_semantics=("arbitrary", "arbitrary"),
        ),
    )(avwt, ssum, ssq, gamma[:, None], beta[:, None])

    return y.reshape(b, c, h, w)


# speedup vs baseline: 1.1086x; 1.1086x over previous
import functools
import math

import jax
import jax.numpy as jnp
from jax.experimental import pallas as pl
from jax.experimental.pallas import tpu as pltpu

BN_EPS = 1e-5
_C2 = 1.4426950408889634 / math.pi
_SUB = 2048


def _main_kernel(xT_ref, w_ref, b_ref,
                 out_ref, ssum_ref, ssq_ref,
                 av_ref, hsq_ref, colsum_ref, deg_ref, xb_ref, xd_ref,
                 *, bn, nb):
    sub = min(_SUB, bn)
    p = pl.program_id(1)
    i = pl.program_id(2)

    @pl.when((p == 0) & (i == 0))
    def _():
        xT = xT_ref[0]
        hsq_ref[...] = 0.5 * jnp.sum(xT * xT, axis=0, keepdims=True)
        xb_ref[...] = xT.astype(jnp.bfloat16)

    @pl.when(p == 0)
    def _():
        parts = []
        for s in range(bn // sub):
            off = i * bn + s * sub
            xb_s = xb_ref[:, pl.ds(off, sub)]
            gram = jax.lax.dot_general(
                xb_s, xb_ref[...], (((0,), (0,)), ((), ())),
                preferred_element_type=jnp.float32)
            hsq_col = hsq_ref[:, pl.ds(off, sub)].T
            z = (gram - hsq_col) - hsq_ref[...]
            av = jnp.exp2(z * _C2)
            av_ref[pl.ds(off, sub), :] = av.astype(jnp.bfloat16)
            parts.append(jnp.sum(av, axis=0, keepdims=True))
        part = parts[0]
        for q in parts[1:]:
            part = part + q

        @pl.when(i == 0)
        def _():
            colsum_ref[...] = part

        @pl.when(i != 0)
        def _():
            colsum_ref[...] += part

    @pl.when(p == 1)
    def _():
        @pl.when(i == 0)
        def _():
            deg = jax.lax.rsqrt(1.0 + colsum_ref[...])
            deg_ref[...] = deg
            xd_ref[...] = (xT_ref[0] * deg).astype(jnp.bfloat16)

        psums, psqs = [], []
        for s in range(bn // sub):
            off = i * bn + s * sub
            av_col = av_ref[:, pl.ds(off, sub)]
            m1t = jax.lax.dot_general(
                xd_ref[...], av_col, (((1,), (0,)), ((), ())),
                preferred_element_type=jnp.float32)
            deg_blk = deg_ref[:, pl.ds(off, sub)]
            xT_blk = xT_ref[0, :, pl.ds(off, sub)]
            aggT = deg_blk * m1t + (deg_blk * deg_blk) * xT_blk
            avwt = jax.lax.dot_general(
                w_ref[...], aggT, (((1,), (0,)), ((), ())),
                preferred_element_type=jnp.float32) + b_ref[...]
            out_ref[0, :, s * sub:(s + 1) * sub] = avwt
            psums.append(jnp.sum(avwt, axis=1, keepdims=True))
            psqs.append(jnp.sum(avwt * avwt, axis=1, keepdims=True))
        psum = psums[0]
        psq = psqs[0]
        for q in psums[1:]:
            psum = psum + q
        for q in psqs[1:]:
            psq = psq + q

        @pl.when(i == 0)
        def _():
            ssum_ref[0] = psum
            ssq_ref[0] = psq

        @pl.when(i != 0)
        def _():
            ssum_ref[0] += psum
            ssq_ref[0] += psq


def _bn_kernel(avwt_ref, ssum_ref, ssq_ref, gamma_ref, beta_ref, out_ref,
               *, count):
    b_total = ssum_ref.shape[0]
    s = ssum_ref[0]
    q = ssq_ref[0]
    for bb in range(1, b_total):
        s = s + ssum_ref[bb]
        q = q + ssq_ref[bb]
    inv = 1.0 / count
    mean = s * inv
    var = q * inv - mean * mean
    scale = gamma_ref[...] * jax.lax.rsqrt(var + BN_EPS)
    shift = beta_ref[...] - mean * scale
    out_ref[0] = avwt_ref[0] * scale + shift


def kernel(x, W, b_lin, gamma, beta):
    b, c, h, w = x.shape
    n = h * w
    bn = min(2048, n)
    nb = n // bn

    xT = x.reshape(b, c, n)

    avwt, ssum, ssq = pl.pallas_call(
        functools.partial(_main_kernel, bn=bn, nb=nb),
        grid=(b, 2, nb),
        in_specs=[
            pl.BlockSpec((1, c, n), lambda bi, p, i: (bi, 0, 0)),
            pl.BlockSpec((c, c), lambda bi, p, i: (0, 0)),
            pl.BlockSpec((c, 1), lambda bi, p, i: (0, 0)),
        ],
        out_specs=[
            pl.BlockSpec((1, c, bn), lambda bi, p, i: (bi, 0, i * p)),
            pl.BlockSpec((1, c, 1), lambda bi, p, i: (bi, 0, 0)),
            pl.BlockSpec((1, c, 1), lambda bi, p, i: (bi, 0, 0)),
        ],
        out_shape=[
            jax.ShapeDtypeStruct((b, c, n), jnp.float32),
            jax.ShapeDtypeStruct((b, c, 1), jnp.float32),
            jax.ShapeDtypeStruct((b, c, 1), jnp.float32),
        ],
        scratch_shapes=[
            pltpu.VMEM((n, n), jnp.bfloat16),
            pltpu.VMEM((1, n), jnp.float32),
            pltpu.VMEM((1, n), jnp.float32),
            pltpu.VMEM((1, n), jnp.float32),
            pltpu.VMEM((c, n), jnp.bfloat16),
            pltpu.VMEM((c, n), jnp.bfloat16),
        ],
        compiler_params=pltpu.CompilerParams(
            dimension_semantics=("arbitrary", "arbitrary", "arbitrary"),
            vmem_limit_bytes=100 * 1024 * 1024,
        ),
    )(xT, W, b_lin[:, None])

    bn3 = min(4096, n)
    y = pl.pallas_call(
        functools.partial(_bn_kernel, count=float(b * n)),
        grid=(b, n // bn3),
        in_specs=[
            pl.BlockSpec((1, c, bn3), lambda bi, i: (bi, 0, i)),
            pl.BlockSpec((b, c, 1), lambda bi, i: (0, 0, 0)),
            pl.BlockSpec((b, c, 1), lambda bi, i: (0, 0, 0)),
            pl.BlockSpec((c, 1), lambda bi, i: (0, 0)),
            pl.BlockSpec((c, 1), lambda bi, i: (0, 0)),
        ],
        out_specs=pl.BlockSpec((1, c, bn3), lambda bi, i: (bi, 0, i)),
        out_shape=jax.ShapeDtypeStruct((b, c, n), jnp.float32),
        compiler_params=pltpu.CompilerParams(
            dimension_semantics=("arbitrary", "arbitrary"),
        ),
    )(avwt, ssum, ssq, gamma[:, None], beta[:, None])

    return y.reshape(b, c, h, w)
